# Initial kernel scaffold; baseline (speedup 1.0000x reference)
import jax
import jax.numpy as jnp
from jax.experimental import pallas as pl


def kernel(XRNA, adj, train_ids, gat1_W, gat1_a, gat2_W, gat2_a, Wh1, bh1, Wloc, bloc, Ws, bs, We1, be1, We2, be2):
    raise NotImplementedError("write your pallas kernel here")


# speedup vs baseline: 4.9349x; 4.9349x over previous
import functools

import jax
import jax.numpy as jnp
from jax import lax
from jax.experimental import pallas as pl
from jax.experimental.pallas import tpu as pltpu
from jax.experimental.pallas import tpu_sc as plsc

_N = 2048
_NFEAT = 512
_NHID = 128
_NS = 64
_NY = 3
_E = 200000
_ALPHA = 0.35

_SC_CORES = 2
_SC_SUBCORES = 16
_NW = _SC_CORES * _SC_SUBCORES

_CHUNK = 128
_EPAD = _NW * _CHUNK * 49
_EBLK = 1024
assert _EPAD % _EBLK == 0


def _softplus(x):
    return jnp.maximum(x, 0.0) + jnp.log1p(jnp.exp(-jnp.abs(x)))


def _proj_body(x_ref, w1_ref, w2_ref, o1_ref, o2_ref):
    x = x_ref[...]
    o1_ref[...] = jnp.dot(x, w1_ref[...], preferred_element_type=jnp.float32)
    o2_ref[...] = jnp.dot(x, w2_ref[...], preferred_element_type=jnp.float32)


def _gat_body(adj_ref, wh1f_ref, wh2f_ref, wh1b_ref, wh2b_ref,
              a1s_ref, a1d_ref, a2s_ref, a2d_ref,
              Wh1_ref, bh1_ref, Wloc_ref, bloc_ref,
              WsT_ref, WsB_ref, bs_ref, We1a_ref, We1b_ref, be1_ref,
              p_ref, q_ref):
    adjb = adj_ref[...]

    def layer(whf, whb, a_src, a_dst):
        f = jnp.dot(whb, a_src, preferred_element_type=jnp.float32)
        g_t = lax.dot_general(a_dst, whf, (((0,), (1,)), ((), ())),
                              preferred_element_type=jnp.float32)
        e = f + g_t
        e = jnp.where(e >= 0, e, _ALPHA * e)
        lgt = jnp.where(adjb > 0, e, jnp.float32(-9e15))
        m = jnp.max(lgt, axis=1, keepdims=True)
        p = jnp.exp(lgt - m)
        att = p / jnp.sum(p, axis=1, keepdims=True)
        xw = jnp.dot(att, whf, preferred_element_type=jnp.float32)
        return jnp.where(xw > 0, xw, jnp.exp(xw) - 1.0)

    x1 = layer(wh1f_ref[...], wh1b_ref[...], a1s_ref[...], a1d_ref[...])
    x2 = layer(wh2f_ref[...], wh2b_ref[...], a2s_ref[...], a2d_ref[...])

    h = _softplus(jnp.dot(x2, Wh1_ref[...], preferred_element_type=jnp.float32)
                  + bh1_ref[...])
    loc = jnp.dot(h, Wloc_ref[...], preferred_element_type=jnp.float32) + bloc_ref[...]
    gv = (jnp.dot(x1, WsT_ref[...], preferred_element_type=jnp.float32)
          + jnp.dot(loc, WsB_ref[...], preferred_element_type=jnp.float32)
          + bs_ref[...])
    gv = jnp.maximum(gv, 0.0)
    p_ref[...] = (jnp.dot(gv, We1a_ref[...], preferred_element_type=jnp.float32)
                  + be1_ref[...])
    q_ref[...] = jnp.dot(gv, We1b_ref[...], preferred_element_type=jnp.float32)


def _sc_gather(p_tab, q_tab, src, dst):
    b_per_w = _EPAD // _NW
    n_chunks = b_per_w // _CHUNK
    mesh = plsc.VectorSubcoreMesh(core_axis_name="c", subcore_axis_name="s")

    @functools.partial(
        pl.kernel, mesh=mesh,
        out_type=(jax.ShapeDtypeStruct((_EPAD, _NHID), jnp.float32),
                  jax.ShapeDtypeStruct((_EPAD, _NHID), jnp.float32)),
        scratch_types=[
            pltpu.VMEM((_CHUNK,), jnp.int32),
            pltpu.VMEM((_CHUNK,), jnp.int32),
            pltpu.VMEM((_CHUNK, _NHID), jnp.float32),
            pltpu.VMEM((_CHUNK, _NHID), jnp.float32),
            pltpu.SemaphoreType.DMA,
            pltpu.SemaphoreType.DMA,
        ],
    )
    def k(p_hbm, q_hbm, src_hbm, dst_hbm, xp_hbm, xq_hbm,
          isv, idv, rp, rq, sem_p, sem_q):
        wid = lax.axis_index("s") * _SC_CORES + lax.axis_index("c")
        base = wid * b_per_w

        def body(i, carry):
            off = base + i * _CHUNK
            pltpu.sync_copy(src_hbm.at[pl.ds(off, _CHUNK)], isv)
            pltpu.sync_copy(dst_hbm.at[pl.ds(off, _CHUNK)], idv)
            cp = pltpu.async_copy(p_hbm.at[isv], rp, sem_p)
            cq = pltpu.async_copy(q_hbm.at[idv], rq, sem_q)
            cp.wait()
            cq.wait()
            pltpu.sync_copy(rp, xp_hbm.at[pl.ds(off, _CHUNK)])
            pltpu.sync_copy(rq, xq_hbm.at[pl.ds(off, _CHUNK)])
            return carry

        lax.fori_loop(0, n_chunks, body, 0)

    return k(p_tab, q_tab, src, dst)


def _edge_body(xp_ref, xq_ref, We2_ref, be2_ref, y_ref, e_ref):
    he = _softplus(xp_ref[...] + xq_ref[...])
    lgt = jnp.dot(he, We2_ref[...], preferred_element_type=jnp.float32) + be2_ref[...]
    m = jnp.max(lgt, axis=1, keepdims=True)
    p = jnp.exp(lgt - m)
    y = p / jnp.sum(p, axis=1, keepdims=True)
    y_ref[...] = y
    mx = jnp.max(y, axis=1, keepdims=True)
    c0 = (y[:, 0:1] == mx).astype(jnp.float32)
    c1 = (y[:, 1:2] == mx).astype(jnp.float32) * (1.0 - c0)
    c2 = (y[:, 2:3] == mx).astype(jnp.float32) * (1.0 - c0) * (1.0 - c1)
    e_ref[...] = jnp.concatenate([c0, c1, c2], axis=1)


def kernel(XRNA, adj, train_ids, gat1_W, gat1_a, gat2_W, gat2_a, Wh1, bh1,
           Wloc, bloc, Ws, bs, We1, be1, We2, be2):
    f32 = jnp.float32

    wh1g, wh2g = pl.pallas_call(
        _proj_body,
        out_shape=(jax.ShapeDtypeStruct((_N, _NHID), f32),
                   jax.ShapeDtypeStruct((_N, _NHID), f32)),
    )(XRNA, gat1_W, gat2_W)

    R = 256
    full = lambda shape: pl.BlockSpec(shape, lambda i: (0, 0))
    blk = lambda shape: pl.BlockSpec(shape, lambda i: (i, 0))
    p_tab, q_tab = pl.pallas_call(
        _gat_body,
        grid=(_N // R,),
        in_specs=[
            blk((R, _N)),
            full((_N, _NHID)),
            full((_N, _NHID)),
            blk((R, _NHID)),
            blk((R, _NHID)),
            full((_NHID, 1)), full((_NHID, 1)),
            full((_NHID, 1)), full((_NHID, 1)),
            full((_NHID, _NHID)), full((1, _NHID)),
            full((_NHID, _NS)), full((1, _NS)),
            full((_NHID, _NS)), full((_NS, _NS)), full((1, _NS)),
            full((_NS, _NHID)), full((_NS, _NHID)), full((1, _NHID)),
        ],
        out_specs=(blk((R, _NHID)), blk((R, _NHID))),
        out_shape=(jax.ShapeDtypeStruct((_N, _NHID), f32),
                   jax.ShapeDtypeStruct((_N, _NHID), f32)),
    )(adj, wh1g, wh2g, wh1g, wh2g,
      gat1_a[:_NHID], gat1_a[_NHID:], gat2_a[:_NHID], gat2_a[_NHID:],
      Wh1, bh1.reshape(1, _NHID), Wloc, bloc.reshape(1, _NS),
      Ws[:_NHID], Ws[_NHID:], bs.reshape(1, _NS),
      We1[:_NS], We1[_NS:], be1.reshape(1, _NHID))

    pad = jnp.zeros((_EPAD - _E,), jnp.int32)
    src = jnp.concatenate([train_ids[:, 0], pad])
    dst = jnp.concatenate([train_ids[:, 1], pad])
    xp, xq = _sc_gather(p_tab, q_tab, src, dst)

    eb = lambda w: pl.BlockSpec((_EBLK, w), lambda i: (i, 0))
    y_pad, e_pad = pl.pallas_call(
        _edge_body,
        grid=(_EPAD // _EBLK,),
        in_specs=[
            eb(_NHID), eb(_NHID),
            full((_NHID, _NY)), full((1, _NY)),
        ],
        out_specs=(eb(_NY), eb(_NY)),
        out_shape=(jax.ShapeDtypeStruct((_EPAD, _NY), f32),
                   jax.ShapeDtypeStruct((_EPAD, _NY), f32)),
    )(xp, xq, We2, be2.reshape(1, _NY))

    return (e_pad[:_E], y_pad[:_E])
